# Initial kernel scaffold; baseline (speedup 1.0000x reference)
import jax
import jax.numpy as jnp
from jax.experimental import pallas as pl


def kernel(features, edge_index, W0, b0, W1, b1, W2, b2):
    raise NotImplementedError("write your pallas kernel here")


# speedup vs baseline: 4.4963x; 4.4963x over previous
import functools

import jax
import jax.numpy as jnp
from jax import lax
from jax.experimental import pallas as pl
from jax.experimental.pallas import tpu as pltpu
from jax.experimental.pallas import tpu_sc as plsc

N = 10000
E = 320000
D = 128
NC = 2
NS = 16
NW = NC * NS
EPW = E // NW
CH = 80
NCHUNK = EPW // CH
RSUB = 624
TAIL = 16
TAIL0 = NS * RSUB
ZR = 104
DW = 128
LANES = 16

_mesh = plsc.VectorSubcoreMesh(core_axis_name="c", subcore_axis_name="s",
                               num_cores=NC, num_subcores=NS)


def _make_hist_kernel(width):
    def body(idx_hbm, hist_hbm, idx_v, ones_v, zrow_v, acc_sh, sem):
        c = lax.axis_index("c")
        s = lax.axis_index("s")
        wid = s * NC + c

        def fill_ones(i, carry):
            def fill_col(j, carry2):
                ones_v[i, pl.ds(j * LANES, LANES)] = jnp.full(
                    (LANES,), 1.0, jnp.float32)
                return carry2

            lax.fori_loop(0, width // LANES, fill_col, 0)
            return carry

        lax.fori_loop(0, CH, fill_ones, 0)

        def fill_zero(i, carry):
            def fill_col(j, carry2):
                zrow_v[i, pl.ds(j * LANES, LANES)] = jnp.zeros(
                    (LANES,), jnp.float32)
                return carry2

            lax.fori_loop(0, width // LANES, fill_col, 0)
            return carry

        lax.fori_loop(0, ZR, fill_zero, 0)

        r0 = s * RSUB
        for k in range(RSUB // ZR):
            pltpu.sync_copy(zrow_v, acc_sh.at[pl.ds(r0 + k * ZR, ZR)])

        @pl.when(s == 0)
        def _zero_tail():
            pltpu.sync_copy(zrow_v.at[pl.ds(0, TAIL)],
                            acc_sh.at[pl.ds(TAIL0, TAIL)])

        plsc.subcore_barrier()

        def body_loop(ci, carry):
            base = wid * EPW + ci * CH
            pltpu.sync_copy(idx_hbm.at[pl.ds(base, CH)], idx_v)
            pltpu.sync_copy(ones_v, acc_sh.at[idx_v], add=True)
            return carry

        lax.fori_loop(0, NCHUNK, body_loop, 0)
        plsc.subcore_barrier()

        pltpu.sync_copy(acc_sh.at[pl.ds(r0, RSUB)],
                        hist_hbm.at[pl.ds(c * N + r0, RSUB)])

        @pl.when(s == 0)
        def _out_tail():
            pltpu.sync_copy(acc_sh.at[pl.ds(TAIL0, TAIL)],
                            hist_hbm.at[pl.ds(c * N + TAIL0, TAIL)])

    return pl.kernel(
        body,
        out_type=jax.ShapeDtypeStruct((NC * N, width), jnp.float32),
        mesh=_mesh,
        scratch_types=[
            pltpu.VMEM((CH,), jnp.int32),
            pltpu.VMEM((CH, width), jnp.float32),
            pltpu.VMEM((ZR, width), jnp.float32),
            pltpu.VMEM_SHARED((N, width), jnp.float32),
            pltpu.SemaphoreType.DMA,
        ],
    )


_hist_kernel = _make_hist_kernel(DW)


_AGG_OUT = jax.ShapeDtypeStruct((NC * N, D), jnp.float32)
_AGG_SCRATCH = [
    pltpu.VMEM((CH,), jnp.int32),
    pltpu.VMEM((CH,), jnp.int32),
    pltpu.VMEM((CH, D), jnp.float32),
    pltpu.VMEM((ZR, D), jnp.float32),
    pltpu.VMEM_SHARED((N, D), jnp.float32),
    pltpu.SemaphoreType.DMA,
]


def _agg_body(y_hbm, src_hbm, dst_hbm, out_hbm,
              src_v, dst_v, rows_v, zrow_v, acc_sh, sem):
    c = lax.axis_index("c")
    s = lax.axis_index("s")
    wid = s * NC + c

    def fill_zero(i, carry):
        def fill_row(j, carry2):
            zrow_v[i, pl.ds(j * LANES, LANES)] = jnp.zeros((LANES,), jnp.float32)
            return carry2

        lax.fori_loop(0, D // LANES, fill_row, 0)
        return carry

    lax.fori_loop(0, ZR, fill_zero, 0)

    r0 = s * RSUB
    for k in range(RSUB // ZR):
        pltpu.sync_copy(zrow_v, acc_sh.at[pl.ds(r0 + k * ZR, ZR)])

    @pl.when(s == 0)
    def _zero_tail():
        pltpu.sync_copy(zrow_v.at[pl.ds(0, TAIL)], acc_sh.at[pl.ds(TAIL0, TAIL)])

    plsc.subcore_barrier()

    def body(ci, carry):
        base = wid * EPW + ci * CH
        pltpu.sync_copy(src_hbm.at[pl.ds(base, CH)], src_v)
        pltpu.sync_copy(dst_hbm.at[pl.ds(base, CH)], dst_v)
        pltpu.async_copy(y_hbm.at[src_v], rows_v, sem).wait()
        pltpu.sync_copy(rows_v, acc_sh.at[dst_v], add=True)
        return carry

    lax.fori_loop(0, NCHUNK, body, 0)
    plsc.subcore_barrier()

    pltpu.sync_copy(acc_sh.at[pl.ds(r0, RSUB)],
                    out_hbm.at[pl.ds(c * N + r0, RSUB)])

    @pl.when(s == 0)
    def _out_tail():
        pltpu.sync_copy(acc_sh.at[pl.ds(TAIL0, TAIL)],
                        out_hbm.at[pl.ds(c * N + TAIL0, TAIL)])


_agg_kernel = pl.kernel(
    _agg_body, out_type=_AGG_OUT, mesh=_mesh, scratch_types=_AGG_SCRATCH)


BLK = 1000
_GRID = N // BLK


def _norm_col(deg_parts):
    deg = deg_parts[0, :, 0:1] + deg_parts[1, :, 0:1]
    return lax.rsqrt(jnp.maximum(deg, 1.0))


def _tc_first_body(x_ref, ds_ref, w_ref, o_ref):
    ns = _norm_col(ds_ref[...])
    o_ref[...] = jnp.dot(x_ref[...] * ns, w_ref[...],
                         preferred_element_type=jnp.float32)


def _tc_mid_body(ap_ref, ds_ref, dd_ref, b_ref, w_ref, o_ref):
    a = ap_ref[0] + ap_ref[1]
    nd = _norm_col(dd_ref[...])
    h = jnp.maximum(a * nd + b_ref[...], 0.0)
    ns = _norm_col(ds_ref[...])
    o_ref[...] = jnp.dot(h * ns, w_ref[...],
                         preferred_element_type=jnp.float32)


def _tc_last_body(ap_ref, dd_ref, b_ref, o_ref):
    a = ap_ref[0] + ap_ref[1]
    nd = _norm_col(dd_ref[...])
    o_ref[...] = a * nd + b_ref[...]


_spec_rows = pl.BlockSpec((BLK, D), lambda i: (i, 0))
_spec_parts = pl.BlockSpec((2, BLK, D), lambda i: (0, i, 0))
_spec_deg = pl.BlockSpec((2, BLK, DW), lambda i: (0, i, 0))
_spec_w = pl.BlockSpec((D, D), lambda i: (0, 0))
_spec_b = pl.BlockSpec((1, D), lambda i: (0, 0))
_out_rows = jax.ShapeDtypeStruct((N, D), jnp.float32)


def _tc_first(x, deg_s, w):
    return pl.pallas_call(
        _tc_first_body,
        grid=(_GRID,),
        in_specs=[_spec_rows, _spec_deg, _spec_w],
        out_specs=_spec_rows,
        out_shape=_out_rows,
    )(x, deg_s, w)


def _tc_mid(agg_parts, deg_s, deg_d, b, w):
    return pl.pallas_call(
        _tc_mid_body,
        grid=(_GRID,),
        in_specs=[_spec_parts, _spec_deg, _spec_deg, _spec_b, _spec_w],
        out_specs=_spec_rows,
        out_shape=_out_rows,
    )(agg_parts, deg_s, deg_d, b, w)


def _tc_last(agg_parts, deg_d, b):
    return pl.pallas_call(
        _tc_last_body,
        grid=(_GRID,),
        in_specs=[_spec_parts, _spec_deg, _spec_b],
        out_specs=_spec_rows,
        out_shape=_out_rows,
    )(agg_parts, deg_d, b)


def kernel(features, edge_index, W0, b0, W1, b1, W2, b2):
    src = edge_index[0]
    dst = edge_index[1]

    deg_s = _hist_kernel(src).reshape(NC, N, DW)
    deg_d = _hist_kernel(dst).reshape(NC, N, DW)

    t0 = _tc_first(features, deg_s, W0)
    a0 = _agg_kernel(t0, src, dst).reshape(NC, N, D)
    t1 = _tc_mid(a0, deg_s, deg_d, b0.reshape(1, D), W1)
    a1 = _agg_kernel(t1, src, dst).reshape(NC, N, D)
    t2 = _tc_mid(a1, deg_s, deg_d, b1.reshape(1, D), W2)
    a2 = _agg_kernel(t2, src, dst).reshape(NC, N, D)
    return _tc_last(a2, deg_d, b2.reshape(1, D))


# speedup vs baseline: 9.5874x; 2.1323x over previous
import functools

import jax
import jax.numpy as jnp
from jax import lax
from jax.experimental import pallas as pl
from jax.experimental.pallas import tpu as pltpu
from jax.experimental.pallas import tpu_sc as plsc

N = 10000
E = 320000
D = 128
NC = 2
NS = 16
NW = NC * NS
EPW = E // NW
CH = 80
NCHUNK = EPW // CH
RSUB = 624
TAIL = 16
TAIL0 = NS * RSUB
ZR = 24
DW = 128
LANES = 16

_mesh = plsc.VectorSubcoreMesh(core_axis_name="c", subcore_axis_name="s",
                               num_cores=NC, num_subcores=NS)


def _hist_body(src_hbm, dst_hbm, hist_hbm,
               sidx_v, didx_v, ones_v, zrow_v, acc_sh, sem_a, sem_b):
    c = lax.axis_index("c")
    s = lax.axis_index("s")
    wid = s * NC + c
    r0 = s * RSUB

    pltpu.sync_copy(src_hbm.at[wid], sidx_v)
    pltpu.sync_copy(dst_hbm.at[wid], didx_v)

    def fill_ones(i, carry):
        def fill_col(j, carry2):
            ones_v[i, pl.ds(j * LANES, LANES)] = jnp.full(
                (LANES,), 1.0, jnp.float32)
            return carry2

        lax.fori_loop(0, DW // LANES, fill_col, 0)
        return carry

    lax.fori_loop(0, CH, fill_ones, 0)

    def fill_zero(i, carry):
        def fill_col(j, carry2):
            zrow_v[i, pl.ds(j * LANES, LANES)] = jnp.zeros(
                (LANES,), jnp.float32)
            return carry2

        lax.fori_loop(0, DW // LANES, fill_col, 0)
        return carry

    lax.fori_loop(0, ZR, fill_zero, 0)

    def zero_acc():
        for k in range(RSUB // ZR):
            pltpu.sync_copy(zrow_v, acc_sh.at[pl.ds(r0 + k * ZR, ZR)])

        @pl.when(s == 0)
        def _zero_tail():
            pltpu.sync_copy(zrow_v.at[pl.ds(0, TAIL)],
                            acc_sh.at[pl.ds(TAIL0, TAIL)])

    def scatter_pass(idx_v):
        pltpu.async_copy(ones_v, acc_sh.at[idx_v.at[0]], sem_a, add=True)

        def wait_a():
            pltpu.make_async_copy(ones_v, acc_sh.at[idx_v.at[0]], sem_a).wait()

        def body_loop(k, carry):
            c0 = 2 * k
            db = pltpu.async_copy(ones_v, acc_sh.at[idx_v.at[c0 + 1]],
                                  sem_b, add=True)
            wait_a()
            pltpu.async_copy(ones_v, acc_sh.at[idx_v.at[c0 + 2]],
                             sem_a, add=True)
            db.wait()
            return carry

        lax.fori_loop(0, NCHUNK // 2, body_loop, 0)
        wait_a()

    def copy_out(h):
        pltpu.sync_copy(acc_sh.at[pl.ds(r0, RSUB)],
                        hist_hbm.at[pl.ds(h * NC * N + c * N + r0, RSUB)])

        @pl.when(s == 0)
        def _out_tail():
            pltpu.sync_copy(acc_sh.at[pl.ds(TAIL0, TAIL)],
                            hist_hbm.at[pl.ds(h * NC * N + c * N + TAIL0, TAIL)])

    zero_acc()
    plsc.subcore_barrier()
    scatter_pass(sidx_v)
    plsc.subcore_barrier()
    copy_out(0)
    zero_acc()
    plsc.subcore_barrier()
    scatter_pass(didx_v)
    plsc.subcore_barrier()
    copy_out(1)


_hist_kernel = pl.kernel(
    _hist_body,
    out_type=jax.ShapeDtypeStruct((2 * NC * N, DW), jnp.float32),
    mesh=_mesh,
    scratch_types=[
        pltpu.VMEM((NCHUNK, CH), jnp.int32),
        pltpu.VMEM((NCHUNK, CH), jnp.int32),
        pltpu.VMEM((CH, DW), jnp.float32),
        pltpu.VMEM((ZR, DW), jnp.float32),
        pltpu.VMEM_SHARED((N, DW), jnp.float32),
        pltpu.SemaphoreType.DMA,
        pltpu.SemaphoreType.DMA,
    ],
)


_AGG_OUT = jax.ShapeDtypeStruct((NC * N, D), jnp.float32)
_AGG_SCRATCH = [
    pltpu.VMEM((EPW,), jnp.int32),
    pltpu.VMEM((NCHUNK, CH), jnp.int32),
    pltpu.VMEM((CH, D), jnp.float32),
    pltpu.VMEM((CH, D), jnp.float32),
    pltpu.VMEM_SHARED((N, D), jnp.float32),
    pltpu.SemaphoreType.DMA,
    pltpu.SemaphoreType.DMA,
]


def _agg_body(y_hbm, src_hbm, dst_hbm, out_hbm,
              sidx_v, didx_v, rows_a, rows_b, acc_sh, sem_a, sem_b):
    c = lax.axis_index("c")
    s = lax.axis_index("s")
    wid = s * NC + c
    r0 = s * RSUB

    pltpu.sync_copy(src_hbm.at[pl.ds(wid * EPW, EPW)], sidx_v)
    pltpu.sync_copy(dst_hbm.at[wid], didx_v)

    def fill_zero(i, carry):
        def fill_row(j, carry2):
            rows_a[i, pl.ds(j * LANES, LANES)] = jnp.zeros((LANES,), jnp.float32)
            return carry2

        lax.fori_loop(0, D // LANES, fill_row, 0)
        return carry

    lax.fori_loop(0, CH, fill_zero, 0)
    for k in range(RSUB // CH):
        pltpu.sync_copy(rows_a, acc_sh.at[pl.ds(r0 + k * CH, CH)])
    _rem = RSUB - (RSUB // CH) * CH
    if _rem:
        pltpu.sync_copy(rows_a.at[pl.ds(0, _rem)],
                        acc_sh.at[pl.ds(r0 + (RSUB // CH) * CH, _rem)])

    @pl.when(s == 0)
    def _zero_tail():
        pltpu.sync_copy(rows_a.at[pl.ds(0, TAIL)], acc_sh.at[pl.ds(TAIL0, TAIL)])

    pltpu.async_copy(y_hbm.at[sidx_v.at[pl.ds(0, CH)]], rows_a, sem_a)
    plsc.subcore_barrier()

    def wait_a():
        pltpu.make_async_copy(y_hbm.at[sidx_v.at[pl.ds(0, CH)]],
                              rows_a, sem_a).wait()

    def body(k, carry):
        c0 = 2 * k
        gb = pltpu.async_copy(
            y_hbm.at[sidx_v.at[pl.ds((c0 + 1) * CH, CH)]], rows_b, sem_b)
        wait_a()
        pltpu.sync_copy(rows_a, acc_sh.at[didx_v.at[c0]], add=True)
        pltpu.async_copy(
            y_hbm.at[sidx_v.at[pl.ds((c0 + 2) * CH, CH)]], rows_a, sem_a)
        gb.wait()
        pltpu.sync_copy(rows_b, acc_sh.at[didx_v.at[c0 + 1]], add=True)
        return carry

    lax.fori_loop(0, NCHUNK // 2, body, 0)
    wait_a()
    pltpu.sync_copy(rows_a, acc_sh.at[didx_v.at[NCHUNK - 1]], add=True)
    plsc.subcore_barrier()

    pltpu.sync_copy(acc_sh.at[pl.ds(r0, RSUB)],
                    out_hbm.at[pl.ds(c * N + r0, RSUB)])

    @pl.when(s == 0)
    def _out_tail():
        pltpu.sync_copy(acc_sh.at[pl.ds(TAIL0, TAIL)],
                        out_hbm.at[pl.ds(c * N + TAIL0, TAIL)])


_agg_kernel = pl.kernel(
    _agg_body, out_type=_AGG_OUT, mesh=_mesh, scratch_types=_AGG_SCRATCH)


BLK = 1000
_GRID = N // BLK


def _norm_col(deg_parts):
    deg = deg_parts[0, :, 0:1] + deg_parts[1, :, 0:1]
    return lax.rsqrt(jnp.maximum(deg, 1.0))


def _tc_first_body(x_ref, ds_ref, w_ref, o_ref):
    ns = _norm_col(ds_ref[...])
    o_ref[...] = jnp.dot(x_ref[...] * ns, w_ref[...],
                         preferred_element_type=jnp.float32)


def _tc_mid_body(ap_ref, ds_ref, dd_ref, b_ref, w_ref, o_ref):
    a = ap_ref[0] + ap_ref[1]
    nd = _norm_col(dd_ref[...])
    h = jnp.maximum(a * nd + b_ref[...], 0.0)
    ns = _norm_col(ds_ref[...])
    o_ref[...] = jnp.dot(h * ns, w_ref[...],
                         preferred_element_type=jnp.float32)


def _tc_last_body(ap_ref, dd_ref, b_ref, o_ref):
    a = ap_ref[0] + ap_ref[1]
    nd = _norm_col(dd_ref[...])
    o_ref[...] = a * nd + b_ref[...]


_spec_rows = pl.BlockSpec((BLK, D), lambda i: (i, 0))
_spec_parts = pl.BlockSpec((2, BLK, D), lambda i: (0, i, 0))
_spec_deg = pl.BlockSpec((2, BLK, DW), lambda i: (0, i, 0))
_spec_w = pl.BlockSpec((D, D), lambda i: (0, 0))
_spec_b = pl.BlockSpec((1, D), lambda i: (0, 0))
_out_rows = jax.ShapeDtypeStruct((N, D), jnp.float32)


def _tc_first(x, deg_s, w):
    return pl.pallas_call(
        _tc_first_body,
        grid=(_GRID,),
        in_specs=[_spec_rows, _spec_deg, _spec_w],
        out_specs=_spec_rows,
        out_shape=_out_rows,
    )(x, deg_s, w)


def _tc_mid(agg_parts, deg_s, deg_d, b, w):
    return pl.pallas_call(
        _tc_mid_body,
        grid=(_GRID,),
        in_specs=[_spec_parts, _spec_deg, _spec_deg, _spec_b, _spec_w],
        out_specs=_spec_rows,
        out_shape=_out_rows,
    )(agg_parts, deg_s, deg_d, b, w)


def _tc_last(agg_parts, deg_d, b):
    return pl.pallas_call(
        _tc_last_body,
        grid=(_GRID,),
        in_specs=[_spec_parts, _spec_deg, _spec_b],
        out_specs=_spec_rows,
        out_shape=_out_rows,
    )(agg_parts, deg_d, b)


def kernel(features, edge_index, W0, b0, W1, b1, W2, b2):
    src = edge_index[0]
    src3 = src.reshape(NW, NCHUNK, CH)
    dst3 = edge_index[1].reshape(NW, NCHUNK, CH)

    hists = _hist_kernel(src3, dst3).reshape(2, NC, N, DW)
    deg_s = hists[0]
    deg_d = hists[1]

    t0 = _tc_first(features, deg_s, W0)
    a0 = _agg_kernel(t0, src, dst3).reshape(NC, N, D)
    t1 = _tc_mid(a0, deg_s, deg_d, b0.reshape(1, D), W1)
    a1 = _agg_kernel(t1, src, dst3).reshape(NC, N, D)
    t2 = _tc_mid(a1, deg_s, deg_d, b1.reshape(1, D), W2)
    a2 = _agg_kernel(t2, src, dst3).reshape(NC, N, D)
    return _tc_last(a2, deg_d, b2.reshape(1, D))
